# baseline (speedup 1.0000x reference)
import jax
import jax.numpy as jnp
from jax import lax
from jax.experimental import pallas as pl
from jax.experimental.pallas import tpu as pltpu
from jax.experimental.pallas import tpu_sc as plsc

N_NODES = 10000
N_EDGES = 320000
D = 128

NC = 2
NS = 16
NW = NC * NS

EDGES_PER_TILE = 10240
E_PAD = NW * EDGES_PER_TILE
CHUNK = 64
PHASES = 5
PROWS = 16
PCHUNK = 32
ACC_ROWS = 10016
ROWS_MAIN = 624
ROWS_LAST = 640


def _sc_body(x_hbm, pk_hbm, out_hbm,
             acc, pk0, pk1, su0, su1, su2, su3, dbuf,
             rows0, rows1, rows2, rows3,
             sem0, sem1, sem2, sem3, psm0, psm1):
    rows = (rows0, rows1, rows2, rows3)
    sems = (sem0, sem1, sem2, sem3)
    sus = (su0, su1, su2, su3)
    pks = (pk0, pk1)
    psms = (psm0, psm1)
    c = lax.axis_index("c")
    s = lax.axis_index("s")
    w = c * NS + s

    def unpack_src(pk, row, half, sref):
        for j in range(CHUNK // 16):
            v = pk[row, pl.ds(64 * half + 16 * j, 16)]
            sref[pl.ds(16 * j, 16)] = v & 65535

    def unpack_dst(pk, row, half):
        for j in range(CHUNK // 16):
            v = pk[row, pl.ds(64 * half + 16 * j, 16)]
            dbuf[pl.ds(16 * j, 16)] = v >> 16

    z16 = jnp.zeros((16,), jnp.float32)

    def zb(i, carry):
        for j in range(D // 16):
            rows0[i, pl.ds(j * 16, 16)] = z16
        return carry

    lax.fori_loop(0, CHUNK, zb, 0)
    row0 = s * ROWS_MAIN
    for r in range(ROWS_LAST // CHUNK):
        pltpu.sync_copy(rows0, acc.at[pl.ds(row0 + r * CHUNK, CHUNK)])

    @pl.when(s == NS - 1)
    def _():
        pltpu.sync_copy(rows0.at[pl.ds(0, ACC_ROWS - N_NODES)],
                        acc.at[pl.ds(N_NODES, ACC_ROWS - N_NODES)])

    plsc.subcore_barrier()

    pltpu.sync_copy(pk_hbm.at[w, 0], pk0)
    pltpu.async_copy(pk_hbm.at[w, 1], pk1, psm1)
    for m in range(3):
        unpack_src(pk0, m // 2, m % 2, sus[m])
        pltpu.async_copy(x_hbm.at[sus[m]], rows[m], sems[m])

    for p in range(PHASES):
        cur = pks[p % 2]
        nxt = pks[(p + 1) % 2]

        def body(k, carry2, cur=cur):
            for j in range(4):
                b = j
                nb = (j + 3) % 4
                pltpu.make_async_copy(
                    x_hbm.at[sus[b]], rows[b], sems[b]).wait()
                unpack_src(cur, 2 * k + (j + 3) // 2, (j + 3) % 2, sus[nb])
                pltpu.async_copy(x_hbm.at[sus[nb]], rows[nb], sems[nb])
                unpack_dst(cur, 2 * k + j // 2, j % 2)
                pltpu.sync_copy(rows[b], acc.at[dbuf], add=True)
            return carry2

        lax.fori_loop(0, (PCHUNK - 4) // 4, body, 0)
        for kk in range(PCHUNK - 4, PCHUNK):
            b = kk % 4
            pltpu.make_async_copy(
                x_hbm.at[sus[b]], rows[b], sems[b]).wait()
            if kk + 3 < PCHUNK:
                nb = (kk + 3) % 4
                unpack_src(cur, (kk + 3) // 2, (kk + 3) % 2, sus[nb])
                pltpu.async_copy(x_hbm.at[sus[nb]], rows[nb], sems[nb])
            elif p + 1 < PHASES:
                m = kk + 3 - PCHUNK
                nb = m % 4
                if m == 0:
                    pltpu.make_async_copy(
                        pk_hbm.at[w, p + 1], nxt, psms[(p + 1) % 2]).wait()
                unpack_src(nxt, m // 2, m % 2, sus[nb])
                pltpu.async_copy(x_hbm.at[sus[nb]], rows[nb], sems[nb])
            unpack_dst(cur, kk // 2, kk % 2)
            pltpu.sync_copy(rows[b], acc.at[dbuf], add=True)
        if p + 2 < PHASES:
            pltpu.async_copy(pk_hbm.at[w, p + 2], cur, psms[p % 2])

    plsc.subcore_barrier()

    @pl.when(s < NS - 1)
    def _():
        pltpu.sync_copy(acc.at[pl.ds(row0, ROWS_MAIN)],
                        out_hbm.at[pl.ds(c * N_NODES + row0, ROWS_MAIN)])

    @pl.when(s == NS - 1)
    def _():
        pltpu.sync_copy(acc.at[pl.ds(row0, ROWS_LAST)],
                        out_hbm.at[pl.ds(c * N_NODES + row0, ROWS_LAST)])


def _sc_scatter(x, pk4):
    mesh = plsc.VectorSubcoreMesh(core_axis_name="c", subcore_axis_name="s")
    f = pl.kernel(
        _sc_body,
        out_type=jax.ShapeDtypeStruct((2 * N_NODES, D), jnp.float32),
        mesh=mesh,
        scratch_types=[
            pltpu.VMEM_SHARED((ACC_ROWS, D), jnp.float32),
            pltpu.VMEM((PROWS, 128), jnp.int32),
            pltpu.VMEM((PROWS, 128), jnp.int32),
            pltpu.VMEM((CHUNK,), jnp.int32),
            pltpu.VMEM((CHUNK,), jnp.int32),
            pltpu.VMEM((CHUNK,), jnp.int32),
            pltpu.VMEM((CHUNK,), jnp.int32),
            pltpu.VMEM((CHUNK,), jnp.int32),
            pltpu.VMEM((CHUNK, D), jnp.float32),
            pltpu.VMEM((CHUNK, D), jnp.float32),
            pltpu.VMEM((CHUNK, D), jnp.float32),
            pltpu.VMEM((CHUNK, D), jnp.float32),
            pltpu.SemaphoreType.DMA,
            pltpu.SemaphoreType.DMA,
            pltpu.SemaphoreType.DMA,
            pltpu.SemaphoreType.DMA,
            pltpu.SemaphoreType.DMA,
            pltpu.SemaphoreType.DMA,
        ],
    )
    return f(x, pk4)


_DNT = (((1,), (1,)), ((), ()))


def _tc_body(p0, p1, xr, wr, wo, bb, o):
    agg = p0[...] + p1[...]
    o[...] = (lax.dot_general(agg, wr[...], _DNT,
                              preferred_element_type=jnp.float32)
              + lax.dot_general(xr[...], wo[...], _DNT,
                                preferred_element_type=jnp.float32)
              + bb[...])


def _tc_combine(partial, x, wrT, woT, b2):
    mb = 1000
    grid = (N_NODES // mb,)
    return pl.pallas_call(
        _tc_body,
        grid=grid,
        in_specs=[
            pl.BlockSpec((mb, D), lambda i: (i, 0)),
            pl.BlockSpec((mb, D), lambda i: (i + N_NODES // mb, 0)),
            pl.BlockSpec((mb, D), lambda i: (i, 0)),
            pl.BlockSpec((D, D), lambda i: (0, 0)),
            pl.BlockSpec((D, D), lambda i: (0, 0)),
            pl.BlockSpec((1, D), lambda i: (0, 0)),
        ],
        out_specs=pl.BlockSpec((mb, D), lambda i: (i, 0)),
        out_shape=jax.ShapeDtypeStruct((N_NODES, D), jnp.float32),
    )(partial, partial, x, wrT, woT, b2)


def kernel(x, edge_index, W_rel, W_root, b):
    ei = edge_index.astype(jnp.int32)
    packed = ei[0] | (ei[1] << 16)
    fill = jnp.full((E_PAD - N_EDGES,), N_NODES << 16, jnp.int32)
    pk4 = jnp.concatenate([packed, fill]).reshape(NW, PHASES, PROWS, 128)
    partial = _sc_scatter(x, pk4)
    return _tc_combine(partial, x, W_rel, W_root, b.reshape(1, D))


# speedup vs baseline: 1.0004x; 1.0004x over previous
import jax
import jax.numpy as jnp
from jax import lax
from jax.experimental import pallas as pl
from jax.experimental.pallas import tpu as pltpu
from jax.experimental.pallas import tpu_sc as plsc

N_NODES = 10000
N_EDGES = 320000
D = 128

NC = 2
NS = 16
NW = NC * NS

EDGES_PER_TILE = 10240
E_PAD = NW * EDGES_PER_TILE
CHUNK = 64
PHASES = 5
PROWS = 16
PCHUNK = 32
ACC_ROWS = 10016
ROWS_MAIN = 624
ROWS_LAST = 640


def _sc_body(x_hbm, pk_hbm, out_hbm,
             acc, pk0, pk1, su0, su1, su2, su3, dbuf,
             rows0, rows1, rows2, rows3,
             sem0, sem1, sem2, sem3, psm0, psm1):
    rows = (rows0, rows1, rows2, rows3)
    sems = (sem0, sem1, sem2, sem3)
    sus = (su0, su1, su2, su3)
    pks = (pk0, pk1)
    psms = (psm0, psm1)
    c = lax.axis_index("c")
    s = lax.axis_index("s")
    w = c * NS + s

    def unpack_src(pk, row, half, sref):
        for j in range(CHUNK // 16):
            v = pk[row, pl.ds(64 * half + 16 * j, 16)]
            sref[pl.ds(16 * j, 16)] = v & 65535

    def unpack_dst(pk, row, half):
        for j in range(CHUNK // 16):
            v = pk[row, pl.ds(64 * half + 16 * j, 16)]
            dbuf[pl.ds(16 * j, 16)] = v >> 16

    z16 = jnp.zeros((16,), jnp.float32)

    def zb(i, carry):
        for j in range(D // 16):
            rows0[i, pl.ds(j * 16, 16)] = z16
        return carry

    lax.fori_loop(0, CHUNK, zb, 0)
    row0 = s * ROWS_MAIN
    for r in range(ROWS_LAST // CHUNK):
        pltpu.sync_copy(rows0, acc.at[pl.ds(row0 + r * CHUNK, CHUNK)])

    @pl.when(s == NS - 1)
    def _():
        pltpu.sync_copy(rows0.at[pl.ds(0, ACC_ROWS - N_NODES)],
                        acc.at[pl.ds(N_NODES, ACC_ROWS - N_NODES)])

    plsc.subcore_barrier()

    pltpu.sync_copy(pk_hbm.at[w, 0], pk0)
    pltpu.async_copy(pk_hbm.at[w, 1], pk1, psm1)
    for m in range(3):
        unpack_src(pk0, m // 2, m % 2, sus[m])
        pltpu.async_copy(x_hbm.at[sus[m]], rows[m], sems[m])

    for p in range(PHASES):
        cur = pks[p % 2]
        nxt = pks[(p + 1) % 2]

        def body(k, carry2, cur=cur):
            for j in range(4):
                b = j
                nb = (j + 3) % 4
                pltpu.make_async_copy(
                    x_hbm.at[sus[b]], rows[b], sems[b]).wait()
                unpack_src(cur, 2 * k + (j + 3) // 2, (j + 3) % 2, sus[nb])
                pltpu.async_copy(x_hbm.at[sus[nb]], rows[nb], sems[nb])
                unpack_dst(cur, 2 * k + j // 2, j % 2)
                pltpu.sync_copy(rows[b], acc.at[dbuf], add=True)
            return carry2

        lax.fori_loop(0, (PCHUNK - 4) // 4, body, 0)
        for kk in range(PCHUNK - 4, PCHUNK):
            b = kk % 4
            pltpu.make_async_copy(
                x_hbm.at[sus[b]], rows[b], sems[b]).wait()
            if kk + 3 < PCHUNK:
                nb = (kk + 3) % 4
                unpack_src(cur, (kk + 3) // 2, (kk + 3) % 2, sus[nb])
                pltpu.async_copy(x_hbm.at[sus[nb]], rows[nb], sems[nb])
            elif p + 1 < PHASES:
                m = kk + 3 - PCHUNK
                nb = m % 4
                if m == 0:
                    pltpu.make_async_copy(
                        pk_hbm.at[w, p + 1], nxt, psms[(p + 1) % 2]).wait()
                unpack_src(nxt, m // 2, m % 2, sus[nb])
                pltpu.async_copy(x_hbm.at[sus[nb]], rows[nb], sems[nb])
            unpack_dst(cur, kk // 2, kk % 2)
            pltpu.sync_copy(rows[b], acc.at[dbuf], add=True)
        if p + 2 < PHASES:
            pltpu.async_copy(pk_hbm.at[w, p + 2], cur, psms[p % 2])

    plsc.subcore_barrier()

    @pl.when(s < NS - 1)
    def _():
        pltpu.sync_copy(acc.at[pl.ds(row0, ROWS_MAIN)],
                        out_hbm.at[pl.ds(c * N_NODES + row0, ROWS_MAIN)])

    @pl.when(s == NS - 1)
    def _():
        pltpu.sync_copy(acc.at[pl.ds(row0, ROWS_LAST)],
                        out_hbm.at[pl.ds(c * N_NODES + row0, ROWS_LAST)])


def _sc_scatter(x, pk4):
    mesh = plsc.VectorSubcoreMesh(core_axis_name="c", subcore_axis_name="s")
    f = pl.kernel(
        _sc_body,
        out_type=jax.ShapeDtypeStruct((2 * N_NODES, D), jnp.float32),
        mesh=mesh,
        scratch_types=[
            pltpu.VMEM_SHARED((ACC_ROWS, D), jnp.float32),
            pltpu.VMEM((PROWS, 128), jnp.int32),
            pltpu.VMEM((PROWS, 128), jnp.int32),
            pltpu.VMEM((CHUNK,), jnp.int32),
            pltpu.VMEM((CHUNK,), jnp.int32),
            pltpu.VMEM((CHUNK,), jnp.int32),
            pltpu.VMEM((CHUNK,), jnp.int32),
            pltpu.VMEM((CHUNK,), jnp.int32),
            pltpu.VMEM((CHUNK, D), jnp.float32),
            pltpu.VMEM((CHUNK, D), jnp.float32),
            pltpu.VMEM((CHUNK, D), jnp.float32),
            pltpu.VMEM((CHUNK, D), jnp.float32),
            pltpu.SemaphoreType.DMA,
            pltpu.SemaphoreType.DMA,
            pltpu.SemaphoreType.DMA,
            pltpu.SemaphoreType.DMA,
            pltpu.SemaphoreType.DMA,
            pltpu.SemaphoreType.DMA,
        ],
    )
    return f(x, pk4)


_DNT = (((1,), (1,)), ((), ()))


def _tc_body(p0, p1, xr, wr, wo, bb, o):
    agg = p0[...] + p1[...]
    o[...] = (lax.dot_general(agg, wr[...], _DNT,
                              preferred_element_type=jnp.float32)
              + lax.dot_general(xr[...], wo[...], _DNT,
                                preferred_element_type=jnp.float32)
              + bb[...])


def _tc_combine(partial, x, wrT, woT, b2):
    mb = 1000
    grid = (N_NODES // mb,)
    return pl.pallas_call(
        _tc_body,
        grid=grid,
        in_specs=[
            pl.BlockSpec((mb, D), lambda i: (i, 0)),
            pl.BlockSpec((mb, D), lambda i: (i + N_NODES // mb, 0)),
            pl.BlockSpec((mb, D), lambda i: (i, 0)),
            pl.BlockSpec((D, D), lambda i: (0, 0)),
            pl.BlockSpec((D, D), lambda i: (0, 0)),
            pl.BlockSpec((1, D), lambda i: (0, 0)),
        ],
        out_specs=pl.BlockSpec((mb, D), lambda i: (i, 0)),
        out_shape=jax.ShapeDtypeStruct((N_NODES, D), jnp.float32),
    )(partial, partial, x, wrT, woT, b2)


def kernel(x, edge_index, W_rel, W_root, b):
    ei = edge_index.astype(jnp.int32)
    packed = ei[0] | (ei[1] << 16)
    fill = (N_NODES + jnp.arange(E_PAD - N_EDGES, dtype=jnp.int32)
            % (ACC_ROWS - N_NODES)) << 16
    pk4 = jnp.concatenate([packed, fill]).reshape(NW, PHASES, PROWS, 128)
    partial = _sc_scatter(x, pk4)
    return _tc_combine(partial, x, W_rel, W_root, b.reshape(1, D))


# speedup vs baseline: 3.1511x; 3.1497x over previous
import jax
import jax.numpy as jnp
from jax import lax
from jax.experimental import pallas as pl
from jax.experimental.pallas import tpu as pltpu
from jax.experimental.pallas import tpu_sc as plsc

N_NODES = 10000
N_EDGES = 320000
D = 128

NC = 2
NS = 16
NW = NC * NS

EDGES_PER_TILE = N_EDGES // NW
CHUNK = 80
NCHUNK = EDGES_PER_TILE // CHUNK
PHASES = 5
PCHUNK = NCHUNK // PHASES
ROWS_MAIN = 624
ROWS_LAST = 640


def _sc_body(x_hbm, pk_hbm, out_hbm,
             acc, pk0, pk1, su0, su1, su2, su3, dbuf,
             rows0, rows1, rows2, rows3,
             sem0, sem1, sem2, sem3, psm0, psm1):
    rows = (rows0, rows1, rows2, rows3)
    sems = (sem0, sem1, sem2, sem3)
    sus = (su0, su1, su2, su3)
    pks = (pk0, pk1)
    psms = (psm0, psm1)
    c = lax.axis_index("c")
    s = lax.axis_index("s")
    w = c * NS + s

    def unpack_src(pk, row, sref):
        for j in range(CHUNK // 16):
            v = pk[row, pl.ds(16 * j, 16)]
            sref[pl.ds(16 * j, 16)] = v & 65535

    def unpack_dst(pk, row):
        for j in range(CHUNK // 16):
            v = pk[row, pl.ds(16 * j, 16)]
            dbuf[pl.ds(16 * j, 16)] = v >> 16

    z16 = jnp.zeros((16,), jnp.float32)

    def zb(i, carry):
        for j in range(D // 16):
            rows0[i, pl.ds(j * 16, 16)] = z16
        return carry

    lax.fori_loop(0, CHUNK, zb, 0)
    row0 = s * ROWS_MAIN
    for r in range(ROWS_LAST // CHUNK):
        pltpu.sync_copy(rows0, acc.at[pl.ds(row0 + r * CHUNK, CHUNK)])
    plsc.subcore_barrier()

    pltpu.sync_copy(pk_hbm.at[w, 0], pk0)
    pltpu.async_copy(pk_hbm.at[w, 1], pk1, psm1)
    for j in range(3):
        unpack_src(pk0, j, sus[j])
        pltpu.async_copy(x_hbm.at[sus[j]], rows[j], sems[j])

    for p in range(PHASES):
        cur = pks[p % 2]
        nxt = pks[(p + 1) % 2]

        def body(k, carry2, p=p, cur=cur):
            for j in range(4):
                kk = 4 * k + j
                b = (p + j) % 4
                nb = (p + j + 3) % 4
                pltpu.make_async_copy(
                    x_hbm.at[sus[b]], rows[b], sems[b]).wait()
                unpack_src(cur, kk + 3, sus[nb])
                pltpu.async_copy(x_hbm.at[sus[nb]], rows[nb], sems[nb])
                unpack_dst(cur, kk)
                pltpu.sync_copy(rows[b], acc.at[dbuf], add=True)
            return carry2

        lax.fori_loop(0, 5, body, 0)
        for kk in range(PCHUNK - 5, PCHUNK):
            b = (p + kk) % 4
            pltpu.make_async_copy(
                x_hbm.at[sus[b]], rows[b], sems[b]).wait()
            if kk + 3 < PCHUNK:
                nb = (p + kk + 3) % 4
                unpack_src(cur, kk + 3, sus[nb])
                pltpu.async_copy(x_hbm.at[sus[nb]], rows[nb], sems[nb])
            elif p + 1 < PHASES:
                nk = kk + 3 - PCHUNK
                nb = (p + 1 + nk) % 4
                if nk == 0:
                    pltpu.make_async_copy(
                        pk_hbm.at[w, p + 1], nxt, psms[(p + 1) % 2]).wait()
                unpack_src(nxt, nk, sus[nb])
                pltpu.async_copy(x_hbm.at[sus[nb]], rows[nb], sems[nb])
            unpack_dst(cur, kk)
            pltpu.sync_copy(rows[b], acc.at[dbuf], add=True)
        if p + 2 < PHASES:
            pltpu.async_copy(pk_hbm.at[w, p + 2], cur, psms[p % 2])

    plsc.subcore_barrier()

    @pl.when(s < NS - 1)
    def _():
        pltpu.sync_copy(acc.at[pl.ds(row0, ROWS_MAIN)],
                        out_hbm.at[pl.ds(c * N_NODES + row0, ROWS_MAIN)])

    @pl.when(s == NS - 1)
    def _():
        pltpu.sync_copy(acc.at[pl.ds(row0, ROWS_LAST)],
                        out_hbm.at[pl.ds(c * N_NODES + row0, ROWS_LAST)])


def _sc_scatter(x, pk4):
    mesh = plsc.VectorSubcoreMesh(core_axis_name="c", subcore_axis_name="s")
    f = pl.kernel(
        _sc_body,
        out_type=jax.ShapeDtypeStruct((2 * N_NODES, D), jnp.float32),
        mesh=mesh,
        scratch_types=[
            pltpu.VMEM_SHARED((N_NODES, D), jnp.float32),
            pltpu.VMEM((PCHUNK, CHUNK), jnp.int32),
            pltpu.VMEM((PCHUNK, CHUNK), jnp.int32),
            pltpu.VMEM((CHUNK,), jnp.int32),
            pltpu.VMEM((CHUNK,), jnp.int32),
            pltpu.VMEM((CHUNK,), jnp.int32),
            pltpu.VMEM((CHUNK,), jnp.int32),
            pltpu.VMEM((CHUNK,), jnp.int32),
            pltpu.VMEM((CHUNK, D), jnp.float32),
            pltpu.VMEM((CHUNK, D), jnp.float32),
            pltpu.VMEM((CHUNK, D), jnp.float32),
            pltpu.VMEM((CHUNK, D), jnp.float32),
            pltpu.SemaphoreType.DMA,
            pltpu.SemaphoreType.DMA,
            pltpu.SemaphoreType.DMA,
            pltpu.SemaphoreType.DMA,
            pltpu.SemaphoreType.DMA,
            pltpu.SemaphoreType.DMA,
        ],
    )
    return f(x, pk4)


_DNT = (((1,), (1,)), ((), ()))


def _tc_body(p0, p1, xr, wr, wo, bb, o):
    agg = p0[...] + p1[...]
    o[...] = (lax.dot_general(agg, wr[...], _DNT,
                              preferred_element_type=jnp.float32)
              + lax.dot_general(xr[...], wo[...], _DNT,
                                preferred_element_type=jnp.float32)
              + bb[...])


def _tc_combine(partial, x, wrT, woT, b2):
    mb = 2000
    grid = (N_NODES // mb,)
    return pl.pallas_call(
        _tc_body,
        grid=grid,
        in_specs=[
            pl.BlockSpec((mb, D), lambda i: (i, 0)),
            pl.BlockSpec((mb, D), lambda i: (i + N_NODES // mb, 0)),
            pl.BlockSpec((mb, D), lambda i: (i, 0)),
            pl.BlockSpec((D, D), lambda i: (0, 0)),
            pl.BlockSpec((D, D), lambda i: (0, 0)),
            pl.BlockSpec((1, D), lambda i: (0, 0)),
        ],
        out_specs=pl.BlockSpec((mb, D), lambda i: (i, 0)),
        out_shape=jax.ShapeDtypeStruct((N_NODES, D), jnp.float32),
    )(partial, partial, x, wrT, woT, b2)


def kernel(x, edge_index, W_rel, W_root, b):
    ei = edge_index.astype(jnp.int32)
    packed = (ei[0] | (ei[1] << 16)).reshape(NW, PHASES, PCHUNK, CHUNK)
    partial = _sc_scatter(x, packed)
    return _tc_combine(partial, x, W_rel, W_root, b.reshape(1, D))
